# baseline (speedup 1.0000x reference)
import jax
import jax.numpy as jnp
from jax import lax
from jax.experimental import pallas as pl
from jax.experimental.pallas import tpu as pltpu
from jax.experimental.pallas import tpu_sc as plsc

NC = 2
NS = 16
L = 16
NW = NC * NS

B = 16384
F = 64
B_PER_W = B // NW
C = 64
N_CHUNKS = B_PER_W // C
N_GROUPS = C // L


def _mf_kernel(uidx_hbm, iidx_hbm, uf_hbm, if_hbm, ub_hbm, ib_hbm, gb_hbm,
               out_hbm,
               uidx_v, iidx_v,
               urows_a, vrows_a, ub_a, ib_a,
               urows_b, vrows_b, ub_b, ib_b,
               gb_v, out_v, acc_v, sem_a, sem_b):
    wid = lax.axis_index("s") * NC + lax.axis_index("c")
    base = wid * B_PER_W

    pltpu.sync_copy(uidx_hbm.at[pl.ds(base, B_PER_W)], uidx_v)
    pltpu.sync_copy(iidx_hbm.at[pl.ds(base, B_PER_W)], iidx_v)
    pltpu.sync_copy(gb_hbm, gb_v.at[pl.ds(0, 1)])

    gb = gb_v[pl.ds(0, L)][0]
    lane = lax.iota(jnp.int32, L)

    bufs = [(urows_a, vrows_a, ub_a, ib_a, sem_a),
            (urows_b, vrows_b, ub_b, ib_b, sem_b)]

    def issue(c):
        urows_v, vrows_v, ub_v, ib_v, sem = bufs[c % 2]
        coff = c * C

        def issue_body(q, _):
            x16 = uidx_v[pl.ds(coff + q * L, L)]
            y16 = iidx_v[pl.ds(coff + q * L, L)]
            for j in range(L):
                k = q * L + j
                x = x16[j]
                y = y16[j]
                pltpu.async_copy(uf_hbm.at[x], urows_v.at[k], sem)
                pltpu.async_copy(if_hbm.at[y], vrows_v.at[k], sem)
                pltpu.async_copy(ub_hbm.at[x], ub_v.at[q * 2 + j // 8, j % 8],
                                 sem)
                pltpu.async_copy(ib_hbm.at[y], ib_v.at[q * 2 + j // 8, j % 8],
                                 sem)
            return 0

        lax.fori_loop(0, C // L, issue_body, 0)

    def drain(c):
        urows_v, vrows_v, ub_v, ib_v, sem = bufs[c % 2]

        def drain_body(k, _):
            pltpu.make_async_copy(uf_hbm.at[0], urows_v.at[0], sem).wait()
            pltpu.make_async_copy(if_hbm.at[0], vrows_v.at[0], sem).wait()
            pltpu.make_async_copy(ub_hbm.at[0], ub_v.at[0, 0], sem).wait()
            pltpu.make_async_copy(ib_hbm.at[0], ib_v.at[0, 0], sem).wait()
            return 0

        lax.fori_loop(0, C, drain_body, 0)

    def compute(c):
        urows_v, vrows_v, ub_v, ib_v, sem = bufs[c % 2]
        coff = c * C

        def group_body(g, _):
            for j in range(L):
                r = g * L + j
                acc = urows_v[r, pl.ds(0, L)] * vrows_v[r, pl.ds(0, L)]
                for k in range(1, F // L):
                    acc = acc + (urows_v[r, pl.ds(k * L, L)]
                                 * vrows_v[r, pl.ds(k * L, L)])
                acc_v[j, pl.ds(0, L)] = acc
            dot = plsc.load_gather(acc_v, [lane, jnp.zeros((L,), jnp.int32)])
            for k in range(1, L):
                dot = dot + plsc.load_gather(
                    acc_v, [lane, jnp.full((L,), k, jnp.int32)])
            goff = g * L
            k16 = goff + lane
            kq16 = jnp.right_shift(k16, 3)
            kr16 = jnp.bitwise_and(k16, 7)
            zero16 = jnp.zeros((L,), jnp.int32)
            bu = plsc.load_gather(ub_v, [kq16, kr16, zero16])
            bi = plsc.load_gather(ib_v, [kq16, kr16, zero16])
            out_v[pl.ds(coff + goff, L)] = dot + bu + bi + gb
            return 0

        lax.fori_loop(0, N_GROUPS, group_body, 0)

    issue(0)
    for c in range(N_CHUNKS):
        if c + 1 < N_CHUNKS:
            issue(c + 1)
        drain(c)
        compute(c)

    pltpu.sync_copy(out_v, out_hbm.at[pl.ds(base, B_PER_W)])


@jax.jit
def _run(user_idx, item_idx, uf, if_, ub, ib, global_bias):
    mesh = plsc.VectorSubcoreMesh(core_axis_name="c", subcore_axis_name="s",
                                  num_cores=NC, num_subcores=NS)
    return pl.kernel(
        _mf_kernel,
        out_type=jax.ShapeDtypeStruct((B,), jnp.float32),
        mesh=mesh,
        scratch_types=[
            pltpu.VMEM((B_PER_W,), jnp.int32),
            pltpu.VMEM((B_PER_W,), jnp.int32),
            pltpu.VMEM((C, F), jnp.float32),
            pltpu.VMEM((C, F), jnp.float32),
            pltpu.VMEM((C // 8, 8, 1), jnp.float32),
            pltpu.VMEM((C // 8, 8, 1), jnp.float32),
            pltpu.VMEM((C, F), jnp.float32),
            pltpu.VMEM((C, F), jnp.float32),
            pltpu.VMEM((C // 8, 8, 1), jnp.float32),
            pltpu.VMEM((C // 8, 8, 1), jnp.float32),
            pltpu.VMEM((L,), jnp.float32),
            pltpu.VMEM((B_PER_W,), jnp.float32),
            pltpu.VMEM((L, L + 1), jnp.float32),
            pltpu.SemaphoreType.DMA,
            pltpu.SemaphoreType.DMA,
        ],
        compiler_params=pltpu.CompilerParams(needs_layout_passes=False),
    )(user_idx, item_idx, uf, if_, ub, ib, global_bias)


def kernel(user_idx, item_idx, user_factors, item_factors, user_bias,
           item_bias, global_bias):
    user_idx = user_idx.astype(jnp.int32)
    item_idx = item_idx.astype(jnp.int32)
    return _run(user_idx, item_idx, user_factors, item_factors, user_bias,
                item_bias, global_bias)


# speedup vs baseline: 1.0020x; 1.0020x over previous
import jax
import jax.numpy as jnp
from jax import lax
from jax.experimental import pallas as pl
from jax.experimental.pallas import tpu as pltpu
from jax.experimental.pallas import tpu_sc as plsc

NC = 2
NS = 16
L = 16
NW = NC * NS

B = 16384
F = 64
B_PER_W = B // NW
C = 64
N_CHUNKS = B_PER_W // C
N_GROUPS = C // L


def _mf_kernel(uidx_hbm, iidx_hbm, uf_hbm, if_hbm, ub_hbm, ib_hbm, gb_hbm,
               out_hbm,
               uidx_v, iidx_v,
               urows_a, vrows_a, ub_a, ib_a,
               urows_b, vrows_b, ub_b, ib_b,
               gb_v, out_v, acc_v,
               sa_u, sa_v, sa_ub, sa_ib, sb_u, sb_v, sb_ub, sb_ib):
    wid = lax.axis_index("s") * NC + lax.axis_index("c")
    base = wid * B_PER_W

    pltpu.sync_copy(uidx_hbm.at[pl.ds(base, B_PER_W)], uidx_v)
    pltpu.sync_copy(iidx_hbm.at[pl.ds(base, B_PER_W)], iidx_v)
    pltpu.sync_copy(gb_hbm, gb_v.at[pl.ds(0, 1)])

    gb = gb_v[pl.ds(0, L)][0]
    lane = lax.iota(jnp.int32, L)

    bufs = [(urows_a, vrows_a, ub_a, ib_a, (sa_u, sa_v, sa_ub, sa_ib)),
            (urows_b, vrows_b, ub_b, ib_b, (sb_u, sb_v, sb_ub, sb_ib))]

    def issue(c):
        urows_v, vrows_v, ub_v, ib_v, sems = bufs[c % 2]
        coff = c * C

        def issue_body(q, _):
            x16 = uidx_v[pl.ds(coff + q * L, L)]
            y16 = iidx_v[pl.ds(coff + q * L, L)]
            for j in range(L):
                k = q * L + j
                x = x16[j]
                y = y16[j]
                pltpu.async_copy(uf_hbm.at[x], urows_v.at[k], sems[0])
                pltpu.async_copy(if_hbm.at[y], vrows_v.at[k], sems[1])
                pltpu.async_copy(ub_hbm.at[x], ub_v.at[q * 2 + j // 8, j % 8],
                                 sems[2])
                pltpu.async_copy(ib_hbm.at[y], ib_v.at[q * 2 + j // 8, j % 8],
                                 sems[3])
            return 0

        lax.fori_loop(0, C // L, issue_body, 0)

    def drain(c):
        urows_v, vrows_v, ub_v, ib_v, sems = bufs[c % 2]

        def drain_body(k, _):
            pltpu.make_async_copy(uf_hbm.at[0], urows_v.at[0], sems[0]).wait()
            pltpu.make_async_copy(if_hbm.at[0], vrows_v.at[0], sems[1]).wait()
            pltpu.make_async_copy(ub_hbm.at[0], ub_v.at[0, 0], sems[2]).wait()
            pltpu.make_async_copy(ib_hbm.at[0], ib_v.at[0, 0], sems[3]).wait()
            return 0

        lax.fori_loop(0, C, drain_body, 0)

    def compute(c):
        urows_v, vrows_v, ub_v, ib_v, sems = bufs[c % 2]
        coff = c * C

        def group_body(g, _):
            for j in range(L):
                r = g * L + j
                acc = urows_v[r, pl.ds(0, L)] * vrows_v[r, pl.ds(0, L)]
                for k in range(1, F // L):
                    acc = acc + (urows_v[r, pl.ds(k * L, L)]
                                 * vrows_v[r, pl.ds(k * L, L)])
                acc_v[j, pl.ds(0, L)] = acc
            dot = plsc.load_gather(acc_v, [lane, jnp.zeros((L,), jnp.int32)])
            for k in range(1, L):
                dot = dot + plsc.load_gather(
                    acc_v, [lane, jnp.full((L,), k, jnp.int32)])
            goff = g * L
            k16 = goff + lane
            kq16 = jnp.right_shift(k16, 3)
            kr16 = jnp.bitwise_and(k16, 7)
            zero16 = jnp.zeros((L,), jnp.int32)
            bu = plsc.load_gather(ub_v, [kq16, kr16, zero16])
            bi = plsc.load_gather(ib_v, [kq16, kr16, zero16])
            out_v[pl.ds(coff + goff, L)] = dot + bu + bi + gb
            return 0

        lax.fori_loop(0, N_GROUPS, group_body, 0)

    issue(0)
    for c in range(N_CHUNKS):
        if c + 1 < N_CHUNKS:
            issue(c + 1)
        drain(c)
        compute(c)

    pltpu.sync_copy(out_v, out_hbm.at[pl.ds(base, B_PER_W)])


@jax.jit
def _run(user_idx, item_idx, uf, if_, ub, ib, global_bias):
    mesh = plsc.VectorSubcoreMesh(core_axis_name="c", subcore_axis_name="s",
                                  num_cores=NC, num_subcores=NS)
    return pl.kernel(
        _mf_kernel,
        out_type=jax.ShapeDtypeStruct((B,), jnp.float32),
        mesh=mesh,
        scratch_types=[
            pltpu.VMEM((B_PER_W,), jnp.int32),
            pltpu.VMEM((B_PER_W,), jnp.int32),
            pltpu.VMEM((C, F), jnp.float32),
            pltpu.VMEM((C, F), jnp.float32),
            pltpu.VMEM((C // 8, 8, 1), jnp.float32),
            pltpu.VMEM((C // 8, 8, 1), jnp.float32),
            pltpu.VMEM((C, F), jnp.float32),
            pltpu.VMEM((C, F), jnp.float32),
            pltpu.VMEM((C // 8, 8, 1), jnp.float32),
            pltpu.VMEM((C // 8, 8, 1), jnp.float32),
            pltpu.VMEM((L,), jnp.float32),
            pltpu.VMEM((B_PER_W,), jnp.float32),
            pltpu.VMEM((L, L + 1), jnp.float32),
            pltpu.SemaphoreType.DMA,
            pltpu.SemaphoreType.DMA,
            pltpu.SemaphoreType.DMA,
            pltpu.SemaphoreType.DMA,
            pltpu.SemaphoreType.DMA,
            pltpu.SemaphoreType.DMA,
            pltpu.SemaphoreType.DMA,
            pltpu.SemaphoreType.DMA,
        ],
        compiler_params=pltpu.CompilerParams(needs_layout_passes=False),
    )(user_idx, item_idx, uf, if_, ub, ib, global_bias)


def kernel(user_idx, item_idx, user_factors, item_factors, user_bias,
           item_bias, global_bias):
    user_idx = user_idx.astype(jnp.int32)
    item_idx = item_idx.astype(jnp.int32)
    return _run(user_idx, item_idx, user_factors, item_factors, user_bias,
                item_bias, global_bias)
